# baseline (speedup 1.0000x reference)
import functools

import jax
import jax.numpy as jnp
from jax import lax
from jax.experimental import pallas as pl
from jax.experimental.pallas import tpu as pltpu
from jax.experimental.pallas import tpu_sc as plsc

_NUM_BUCKETS = 32
_NUM_HEADS = 16
_Q_LEN = 2048
_K_LEN = 2048

_NSHIFT = 16
_LUT_W = 4096
_LINE_W = 4368
_ROWS_PER_WORKER = _Q_LEN // 2
_DMAS_PER_WORKER = _ROWS_PER_WORKER // _NSHIFT
_WINDOW = 8

_THRESHOLDS = (19, 21, 24, 27, 31, 35, 40, 46, 52, 59, 67, 77, 87, 99, 113)


def _sc_body(table_hbm, out_hbm, tab_v, line_v, lut_v, sem):
    head = lax.axis_index("s")
    half = lax.axis_index("c")
    pltpu.sync_copy(table_hbm, tab_v)

    lane = lax.broadcasted_iota(jnp.int32, (16,), 0)
    h16 = jnp.full((16,), head, jnp.int32)

    def line_chunk(t, _):
        d = t * 16 + lane - (_Q_LEN - 1)
        n = jnp.abs(d)
        large = jnp.full((16,), _NUM_BUCKETS // 2, jnp.int32)
        for tj in _THRESHOLDS:
            large = large + (n >= tj).astype(jnp.int32)
        neg_b = jnp.where(n < _NUM_BUCKETS // 2, n, large)
        pos_b = _NUM_BUCKETS // 2 + jnp.minimum(d, _NUM_BUCKETS // 2 - 1)
        b = jnp.where(d > 0, pos_b, neg_b)
        line_v[pl.ds(t * 16, 16)] = plsc.load_gather(tab_v, [b, h16])
        return 0

    lax.fori_loop(0, _LINE_W // 16, line_chunk, 0)

    def copy_chunk(t, _):
        base = t * 16
        for c in range(_NSHIFT):
            lut_v[c, pl.ds(base, 16)] = plsc.load_gather(
                line_v, [lane + (base + (_NSHIFT - 1 - c))]
            )
        return 0

    lax.fori_loop(0, _LUT_W // 16, copy_chunk, 0)

    descs = []
    for t in range(_DMAS_PER_WORKER):
        q0 = half * _ROWS_PER_WORKER + t * _NSHIFT
        off = (_Q_LEN - _NSHIFT) - q0
        cp = pltpu.make_async_copy(
            lut_v.at[:, pl.ds(off, _K_LEN)],
            out_hbm.at[pl.ds(head * _Q_LEN + q0, _NSHIFT), :],
            sem,
        )
        cp.start()
        descs.append(cp)
        if t >= _WINDOW:
            descs[t - _WINDOW].wait()
    for t in range(_DMAS_PER_WORKER - _WINDOW, _DMAS_PER_WORKER):
        descs[t].wait()


def _expand(table):
    mesh = plsc.VectorSubcoreMesh(core_axis_name="c", subcore_axis_name="s")
    run = functools.partial(
        pl.kernel,
        mesh=mesh,
        out_type=jax.ShapeDtypeStruct((_NUM_HEADS * _Q_LEN, _K_LEN), jnp.float32),
        scratch_types=[
            pltpu.VMEM((_NUM_BUCKETS, _NUM_HEADS), jnp.float32),
            pltpu.VMEM((_LINE_W,), jnp.float32),
            pltpu.VMEM((_NSHIFT, _LUT_W), jnp.float32),
            pltpu.SemaphoreType.DMA,
        ],
        compiler_params=pltpu.CompilerParams(
            use_tc_tiling_on_sc=False, needs_layout_passes=False
        ),
    )(_sc_body)
    return run(table)


def kernel(q_len, k_len, table):
    del q_len, k_len
    flat = _expand(table)
    return flat.reshape(_NUM_HEADS, _Q_LEN, _K_LEN)


# speedup vs baseline: 1.0045x; 1.0045x over previous
import functools

import jax
import jax.numpy as jnp
from jax import lax
from jax.experimental import pallas as pl
from jax.experimental.pallas import tpu as pltpu
from jax.experimental.pallas import tpu_sc as plsc

_NUM_BUCKETS = 32
_NUM_HEADS = 16
_Q_LEN = 2048
_K_LEN = 2048

_NSHIFT = 16
_LUT_W = 4096
_LINE_W = 4368
_ROWS_PER_WORKER = _Q_LEN // 2
_DMAS_PER_WORKER = _ROWS_PER_WORKER // _NSHIFT
_WINDOW = 8

_THRESHOLDS = (19, 21, 24, 27, 31, 35, 40, 46, 52, 59, 67, 77, 87, 99, 113)


def _sc_body(table_hbm, out_hbm, tab_v, line_v, lut_v, sem):
    head = lax.axis_index("s")
    half = lax.axis_index("c")
    pltpu.sync_copy(table_hbm, tab_v)

    lane = lax.broadcasted_iota(jnp.int32, (16,), 0)
    h16 = jnp.full((16,), head, jnp.int32)

    def line_chunk(t, _):
        d = t * 16 + lane - (_Q_LEN - 1)
        n = jnp.abs(d)
        large = jnp.full((16,), _NUM_BUCKETS // 2, jnp.int32)
        for tj in _THRESHOLDS:
            large = large + (n >= tj).astype(jnp.int32)
        neg_b = jnp.where(n < _NUM_BUCKETS // 2, n, large)
        pos_b = _NUM_BUCKETS // 2 + jnp.minimum(d, _NUM_BUCKETS // 2 - 1)
        b = jnp.where(d > 0, pos_b, neg_b)
        line_v[pl.ds(t * 16, 16)] = plsc.load_gather(tab_v, [b, h16])
        return 0

    lax.fori_loop(0, _LINE_W // 16, line_chunk, 0)

    def copy_chunk(t, _):
        base = t * 16
        for c in range(_NSHIFT):
            lut_v[c, pl.ds(base, 16)] = plsc.load_gather(
                line_v, [lane + (base + (_NSHIFT - 1 - c))]
            )
        return 0

    lax.fori_loop(0, _LUT_W // 16, copy_chunk, 0)

    descs = []
    for t in range(_DMAS_PER_WORKER):
        q0 = half * _ROWS_PER_WORKER + t * _NSHIFT
        off = (_Q_LEN - _NSHIFT) - q0
        cp = pltpu.make_async_copy(
            lut_v.at[:, pl.ds(off, _K_LEN)],
            out_hbm.at[head, pl.ds(q0, _NSHIFT), :],
            sem,
        )
        cp.start()
        descs.append(cp)
        if t >= _WINDOW:
            descs[t - _WINDOW].wait()
    for t in range(_DMAS_PER_WORKER - _WINDOW, _DMAS_PER_WORKER):
        descs[t].wait()


def _expand(table):
    mesh = plsc.VectorSubcoreMesh(core_axis_name="c", subcore_axis_name="s")
    run = functools.partial(
        pl.kernel,
        mesh=mesh,
        out_type=jax.ShapeDtypeStruct((_NUM_HEADS, _Q_LEN, _K_LEN), jnp.float32),
        scratch_types=[
            pltpu.VMEM((_NUM_BUCKETS, _NUM_HEADS), jnp.float32),
            pltpu.VMEM((_LINE_W,), jnp.float32),
            pltpu.VMEM((_NSHIFT, _LUT_W), jnp.float32),
            pltpu.SemaphoreType.DMA,
        ],
        compiler_params=pltpu.CompilerParams(
            use_tc_tiling_on_sc=False, needs_layout_passes=False
        ),
    )(_sc_body)
    return run(table)


def kernel(q_len, k_len, table):
    del q_len, k_len
    return _expand(table)


# speedup vs baseline: 2.7843x; 2.7718x over previous
import functools

import jax
import jax.numpy as jnp
from jax import lax
from jax.experimental import pallas as pl
from jax.experimental.pallas import tpu as pltpu
from jax.experimental.pallas import tpu_sc as plsc

_NUM_BUCKETS = 32
_NUM_HEADS = 16
_Q_LEN = 2048
_K_LEN = 2048

_QT = 8
_PATCH_W = 256
_NPATCH = 32
_LB_W = 512
_TILES_PER_WORKER = _Q_LEN // 2 // _QT
_ROUND = 16

_THRESHOLDS = (19, 21, 24, 27, 31, 35, 40, 46, 52, 59, 67, 77, 87, 99, 113)


def _sc_body(table_hbm, out_hbm, tab_v, lb_v, const_v, patch_v, fsem, psem):
    head = lax.axis_index("s")
    half = lax.axis_index("c")
    pltpu.sync_copy(table_hbm, tab_v)

    lane = lax.broadcasted_iota(jnp.int32, (16,), 0)
    h16 = jnp.full((16,), head, jnp.int32)
    v31 = plsc.load_gather(
        tab_v, [jnp.full((16,), _NUM_BUCKETS - 1, jnp.int32), h16]
    )

    def line_chunk(t, _):
        d = t * 16 + lane - (_LB_W // 2 - 1)
        n = jnp.abs(d)
        large = jnp.full((16,), _NUM_BUCKETS // 2, jnp.int32)
        for tj in _THRESHOLDS:
            large = large + (n >= tj).astype(jnp.int32)
        neg_b = jnp.where(n < _NUM_BUCKETS // 2, n, large)
        pos_b = _NUM_BUCKETS // 2 + jnp.minimum(d, _NUM_BUCKETS // 2 - 1)
        b = jnp.where(d > 0, pos_b, neg_b)
        lb_v[t, :] = plsc.load_gather(tab_v, [b, h16])
        return 0

    lax.fori_loop(0, _LB_W // 16, line_chunk, 0)

    def fill_chunk(t, _):
        const_v[t % _QT, pl.ds((t // _QT) * 16, 16)] = v31
        return 0

    lax.fori_loop(0, _QT * _K_LEN // 16, fill_chunk, 0)

    def patch_chunk(t, _):
        si = t // (_QT * _PATCH_W // 16)
        rem = t % (_QT * _PATCH_W // 16)
        r = rem // (_PATCH_W // 16)
        m = rem % (_PATCH_W // 16)
        j = (_LB_W // 2 - 1) - 8 * si - r + 16 * m + lane
        vals = plsc.load_gather(lb_v, [j >> 4, j & 15])
        patch_v[si, r, pl.ds(m * 16, 16)] = vals
        return 0

    lax.fori_loop(0, _NPATCH * _QT * _PATCH_W // 16, patch_chunk, 0)

    fills, patches = [], []

    def mk(t):
        q0 = half * (_Q_LEN // 2) + t * _QT
        a = jnp.clip((q0 - 112) & -128, 0, _K_LEN - _PATCH_W)
        si = (q0 - a) >> 3
        q0 = pl.multiple_of(q0, _QT)
        a = pl.multiple_of(a, 128)
        fill = pltpu.make_async_copy(
            const_v, out_hbm.at[head, pl.ds(q0, _QT), :], fsem
        )
        patch = pltpu.make_async_copy(
            patch_v.at[si],
            out_hbm.at[head, pl.ds(q0, _QT), pl.ds(a, _PATCH_W)],
            psem,
        )
        return fill, patch

    for t in range(_TILES_PER_WORKER):
        f, p = mk(t)
        fills.append(f)
        patches.append(p)
    for base in range(0, _TILES_PER_WORKER, _ROUND):
        for j in range(_ROUND):
            fills[base + j].start()
        for j in range(_ROUND):
            fills[base + j].wait()
        for j in range(_ROUND):
            patches[base + j].start()
    for p in patches:
        p.wait()


def _expand(table):
    mesh = plsc.VectorSubcoreMesh(core_axis_name="c", subcore_axis_name="s")
    run = functools.partial(
        pl.kernel,
        mesh=mesh,
        out_type=jax.ShapeDtypeStruct((_NUM_HEADS, _Q_LEN, _K_LEN), jnp.float32),
        scratch_types=[
            pltpu.VMEM((_NUM_BUCKETS, _NUM_HEADS), jnp.float32),
            pltpu.VMEM((_LB_W // 16, 16), jnp.float32),
            pltpu.VMEM((_QT, _K_LEN), jnp.float32),
            pltpu.VMEM((_NPATCH, _QT, _PATCH_W), jnp.float32),
            pltpu.SemaphoreType.DMA,
            pltpu.SemaphoreType.DMA,
        ],
        compiler_params=pltpu.CompilerParams(
            use_tc_tiling_on_sc=True, needs_layout_passes=False
        ),
    )(_sc_body)
    return run(table)


def kernel(q_len, k_len, table):
    del q_len, k_len
    return _expand(table)


# speedup vs baseline: 3.0534x; 1.0966x over previous
import functools

import jax
import jax.numpy as jnp
from jax import lax
from jax.experimental import pallas as pl
from jax.experimental.pallas import tpu as pltpu
from jax.experimental.pallas import tpu_sc as plsc

_NUM_BUCKETS = 32
_NUM_HEADS = 16
_Q_LEN = 2048
_K_LEN = 2048

_QT = 8
_PATCH_W = 256
_NPATCH = 32
_LB_W = 512
_TILES_PER_WORKER = _Q_LEN // 2 // _QT
_ROUND = 16

_THRESHOLDS = (19, 21, 24, 27, 31, 35, 40, 46, 52, 59, 67, 77, 87, 99, 113)


def _sc_body(table_hbm, out_hbm, tab_v, lb_v, const_v, patch_v, fsa, fsb, psem):
    head = lax.axis_index("s")
    half = lax.axis_index("c")
    pltpu.sync_copy(table_hbm, tab_v)

    lane = lax.broadcasted_iota(jnp.int32, (16,), 0)
    h16 = jnp.full((16,), head, jnp.int32)
    v31 = plsc.load_gather(
        tab_v, [jnp.full((16,), _NUM_BUCKETS - 1, jnp.int32), h16]
    )

    def fill_chunk(t, _):
        const_v[t % _QT, pl.ds((t // _QT) * 16, 16)] = v31
        return 0

    lax.fori_loop(0, _QT * _K_LEN // 16, fill_chunk, 0)

    fills, patches = [], []

    def mk(t):
        q0 = half * (_Q_LEN // 2) + t * _QT
        a = jnp.clip((q0 - 112) & -128, 0, _K_LEN - _PATCH_W)
        si = (q0 - a) >> 3
        q0 = pl.multiple_of(q0, _QT)
        a = pl.multiple_of(a, 128)
        fill = pltpu.make_async_copy(
            const_v,
            out_hbm.at[head, pl.ds(q0, _QT), :],
            fsa if (t // _ROUND) % 2 == 0 else fsb,
        )
        patch = pltpu.make_async_copy(
            patch_v.at[si],
            out_hbm.at[head, pl.ds(q0, _QT), pl.ds(a, _PATCH_W)],
            psem,
        )
        return fill, patch

    for t in range(_TILES_PER_WORKER):
        f, p = mk(t)
        fills.append(f)
        patches.append(p)

    for j in range(_ROUND):
        fills[j].start()

    def line_chunk(t, _):
        d = t * 16 + lane - (_LB_W // 2 - 1)
        n = jnp.abs(d)
        large = jnp.full((16,), _NUM_BUCKETS // 2, jnp.int32)
        for tj in _THRESHOLDS:
            large = large + (n >= tj).astype(jnp.int32)
        neg_b = jnp.where(n < _NUM_BUCKETS // 2, n, large)
        pos_b = _NUM_BUCKETS // 2 + jnp.minimum(d, _NUM_BUCKETS // 2 - 1)
        b = jnp.where(d > 0, pos_b, neg_b)
        lb_v[t, :] = plsc.load_gather(tab_v, [b, h16])
        return 0

    lax.fori_loop(0, _LB_W // 16, line_chunk, 0)

    def patch_row(t, _):
        si = t // _QT
        r = t % _QT
        j0 = (_LB_W // 2 - 1) - 8 * si - r + lane
        for m in range(_PATCH_W // 16):
            j = j0 + 16 * m
            patch_v[si, r, pl.ds(m * 16, 16)] = plsc.load_gather(
                lb_v, [j >> 4, j & 15]
            )
        return 0

    lax.fori_loop(0, _NPATCH * _QT, patch_row, 0)

    nrounds = _TILES_PER_WORKER // _ROUND
    for k in range(nrounds):
        if k + 1 < nrounds:
            for j in range(_ROUND):
                fills[(k + 1) * _ROUND + j].start()
        for j in range(_ROUND):
            fills[k * _ROUND + j].wait()
        for j in range(_ROUND):
            patches[k * _ROUND + j].start()
    for p in patches:
        p.wait()


def _expand(table):
    mesh = plsc.VectorSubcoreMesh(core_axis_name="c", subcore_axis_name="s")
    run = functools.partial(
        pl.kernel,
        mesh=mesh,
        out_type=jax.ShapeDtypeStruct((_NUM_HEADS, _Q_LEN, _K_LEN), jnp.float32),
        scratch_types=[
            pltpu.VMEM((_NUM_BUCKETS, _NUM_HEADS), jnp.float32),
            pltpu.VMEM((_LB_W // 16, 16), jnp.float32),
            pltpu.VMEM((_QT, _K_LEN), jnp.float32),
            pltpu.VMEM((_NPATCH, _QT, _PATCH_W), jnp.float32),
            pltpu.SemaphoreType.DMA,
            pltpu.SemaphoreType.DMA,
            pltpu.SemaphoreType.DMA,
        ],
        compiler_params=pltpu.CompilerParams(
            use_tc_tiling_on_sc=True, needs_layout_passes=False
        ),
    )(_sc_body)
    return run(table)


def kernel(q_len, k_len, table):
    del q_len, k_len
    return _expand(table)


# speedup vs baseline: 3.2334x; 1.0590x over previous
import functools

import jax
import jax.numpy as jnp
from jax import lax
from jax.experimental import pallas as pl
from jax.experimental.pallas import tpu as pltpu
from jax.experimental.pallas import tpu_sc as plsc

_NUM_BUCKETS = 32
_NUM_HEADS = 16
_Q_LEN = 2048
_K_LEN = 2048

_QT = 8
_PATCH_W = 256
_NPATCH = 32
_LB_W = 512
_TILES_PER_WORKER = _Q_LEN // 2 // _QT
_ROUND = 32

_THRESHOLDS = (19, 21, 24, 27, 31, 35, 40, 46, 52, 59, 67, 77, 87, 99, 113)


def _sc_body(table_hbm, out_hbm, tab_v, lb_v, const_v, patch_v, fsa, fsb, psem):
    head = lax.axis_index("s")
    half = lax.axis_index("c")
    pltpu.sync_copy(table_hbm, tab_v)

    lane = lax.broadcasted_iota(jnp.int32, (16,), 0)
    h16 = jnp.full((16,), head, jnp.int32)
    v31 = plsc.load_gather(
        tab_v, [jnp.full((16,), _NUM_BUCKETS - 1, jnp.int32), h16]
    )

    def fill_chunk(t, _):
        const_v[t % _QT, pl.ds((t // _QT) * 16, 16)] = v31
        return 0

    lax.fori_loop(0, _QT * _K_LEN // 16, fill_chunk, 0)

    fills, patches = [], []

    def mk(t):
        q0 = half * (_Q_LEN // 2) + t * _QT
        a = jnp.clip((q0 - 112) & -128, 0, _K_LEN - _PATCH_W)
        si = (q0 - a) >> 3
        q0 = pl.multiple_of(q0, _QT)
        a = pl.multiple_of(a, 128)
        fill = pltpu.make_async_copy(
            const_v,
            out_hbm.at[head, pl.ds(q0, _QT), :],
            fsa if (t // _ROUND) % 2 == 0 else fsb,
        )
        patch = pltpu.make_async_copy(
            patch_v.at[si],
            out_hbm.at[head, pl.ds(q0, _QT), pl.ds(a, _PATCH_W)],
            psem,
        )
        return fill, patch

    for t in range(_TILES_PER_WORKER):
        f, p = mk(t)
        fills.append(f)
        patches.append(p)

    for j in range(_ROUND):
        fills[j].start()

    def line_chunk(t, _):
        d = t * 16 + lane - (_LB_W // 2 - 1)
        n = jnp.abs(d)
        large = jnp.full((16,), _NUM_BUCKETS // 2, jnp.int32)
        for tj in _THRESHOLDS:
            large = large + (n >= tj).astype(jnp.int32)
        neg_b = jnp.where(n < _NUM_BUCKETS // 2, n, large)
        pos_b = _NUM_BUCKETS // 2 + jnp.minimum(d, _NUM_BUCKETS // 2 - 1)
        b = jnp.where(d > 0, pos_b, neg_b)
        lb_v[t, :] = plsc.load_gather(tab_v, [b, h16])
        return 0

    lax.fori_loop(0, _LB_W // 16, line_chunk, 0)

    def patch_row(t, _):
        si = t // _QT
        r = t % _QT
        j0 = (_LB_W // 2 - 1) - 8 * si - r + lane
        for m in range(_PATCH_W // 16):
            j = j0 + 16 * m
            patch_v[si, r, pl.ds(m * 16, 16)] = plsc.load_gather(
                lb_v, [j >> 4, j & 15]
            )
        return 0

    lax.fori_loop(0, _NPATCH * _QT, patch_row, 0)

    nrounds = _TILES_PER_WORKER // _ROUND
    for k in range(nrounds):
        if k + 1 < nrounds:
            for j in range(_ROUND):
                fills[(k + 1) * _ROUND + j].start()
        for j in range(_ROUND):
            fills[k * _ROUND + j].wait()
        for j in range(_ROUND):
            patches[k * _ROUND + j].start()
    for p in patches:
        p.wait()


def _expand(table):
    mesh = plsc.VectorSubcoreMesh(core_axis_name="c", subcore_axis_name="s")
    run = functools.partial(
        pl.kernel,
        mesh=mesh,
        out_type=jax.ShapeDtypeStruct((_NUM_HEADS, _Q_LEN, _K_LEN), jnp.float32),
        scratch_types=[
            pltpu.VMEM((_NUM_BUCKETS, _NUM_HEADS), jnp.float32),
            pltpu.VMEM((_LB_W // 16, 16), jnp.float32),
            pltpu.VMEM((_QT, _K_LEN), jnp.float32),
            pltpu.VMEM((_NPATCH, _QT, _PATCH_W), jnp.float32),
            pltpu.SemaphoreType.DMA,
            pltpu.SemaphoreType.DMA,
            pltpu.SemaphoreType.DMA,
        ],
        compiler_params=pltpu.CompilerParams(
            use_tc_tiling_on_sc=True, needs_layout_passes=False
        ),
    )(_sc_body)
    return run(table)


def kernel(q_len, k_len, table):
    del q_len, k_len
    return _expand(table)


# speedup vs baseline: 3.2561x; 1.0070x over previous
import functools

import jax
import jax.numpy as jnp
from jax import lax
from jax.experimental import pallas as pl
from jax.experimental.pallas import tpu as pltpu
from jax.experimental.pallas import tpu_sc as plsc

_NUM_BUCKETS = 32
_NUM_HEADS = 16
_Q_LEN = 2048
_K_LEN = 2048

_QT = 8
_PATCH_W = 256
_NPATCH = 32
_LB_W = 512
_TILES_PER_WORKER = _Q_LEN // 2 // _QT
_ROUND = 64

_THRESHOLDS = (19, 21, 24, 27, 31, 35, 40, 46, 52, 59, 67, 77, 87, 99, 113)


def _sc_body(table_hbm, out_hbm, tab_v, lb_v, const_v, patch_v, fsa, fsb, psem):
    head = lax.axis_index("s")
    half = lax.axis_index("c")
    pltpu.sync_copy(table_hbm, tab_v)

    lane = lax.broadcasted_iota(jnp.int32, (16,), 0)
    h16 = jnp.full((16,), head, jnp.int32)
    v31 = plsc.load_gather(
        tab_v, [jnp.full((16,), _NUM_BUCKETS - 1, jnp.int32), h16]
    )

    def fill_chunk(t, _):
        const_v[t % _QT, pl.ds((t // _QT) * 16, 16)] = v31
        return 0

    lax.fori_loop(0, _QT * _K_LEN // 16, fill_chunk, 0)

    fills, patches = [], []

    def mk(t):
        q0 = half * (_Q_LEN // 2) + t * _QT
        a = jnp.clip((q0 - 112) & -128, 0, _K_LEN - _PATCH_W)
        si = (q0 - a) >> 3
        q0 = pl.multiple_of(q0, _QT)
        a = pl.multiple_of(a, 128)
        fill = pltpu.make_async_copy(
            const_v,
            out_hbm.at[head, pl.ds(q0, _QT), :],
            fsa if (t // _ROUND) % 2 == 0 else fsb,
        )
        patch = pltpu.make_async_copy(
            patch_v.at[si],
            out_hbm.at[head, pl.ds(q0, _QT), pl.ds(a, _PATCH_W)],
            psem,
        )
        return fill, patch

    for t in range(_TILES_PER_WORKER):
        f, p = mk(t)
        fills.append(f)
        patches.append(p)

    for j in range(_ROUND):
        fills[j].start()

    def line_chunk(t, _):
        d = t * 16 + lane - (_LB_W // 2 - 1)
        n = jnp.abs(d)
        large = jnp.full((16,), _NUM_BUCKETS // 2, jnp.int32)
        for tj in _THRESHOLDS:
            large = large + (n >= tj).astype(jnp.int32)
        neg_b = jnp.where(n < _NUM_BUCKETS // 2, n, large)
        pos_b = _NUM_BUCKETS // 2 + jnp.minimum(d, _NUM_BUCKETS // 2 - 1)
        b = jnp.where(d > 0, pos_b, neg_b)
        lb_v[t, :] = plsc.load_gather(tab_v, [b, h16])
        return 0

    lax.fori_loop(0, _LB_W // 16, line_chunk, 0)

    def patch_row(t, _):
        si = t // _QT
        r = t % _QT
        j0 = (_LB_W // 2 - 1) - 8 * si - r + lane
        for m in range(_PATCH_W // 16):
            j = j0 + 16 * m
            patch_v[si, r, pl.ds(m * 16, 16)] = plsc.load_gather(
                lb_v, [j >> 4, j & 15]
            )
        return 0

    lax.fori_loop(0, _NPATCH * _QT, patch_row, 0)

    nrounds = _TILES_PER_WORKER // _ROUND
    for k in range(nrounds):
        if k + 1 < nrounds:
            for j in range(_ROUND):
                fills[(k + 1) * _ROUND + j].start()
        for j in range(_ROUND):
            fills[k * _ROUND + j].wait()
        for j in range(_ROUND):
            patches[k * _ROUND + j].start()
    for p in patches:
        p.wait()


def _expand(table):
    mesh = plsc.VectorSubcoreMesh(core_axis_name="c", subcore_axis_name="s")
    run = functools.partial(
        pl.kernel,
        mesh=mesh,
        out_type=jax.ShapeDtypeStruct((_NUM_HEADS, _Q_LEN, _K_LEN), jnp.float32),
        scratch_types=[
            pltpu.VMEM((_NUM_BUCKETS, _NUM_HEADS), jnp.float32),
            pltpu.VMEM((_LB_W // 16, 16), jnp.float32),
            pltpu.VMEM((_QT, _K_LEN), jnp.float32),
            pltpu.VMEM((_NPATCH, _QT, _PATCH_W), jnp.float32),
            pltpu.SemaphoreType.DMA,
            pltpu.SemaphoreType.DMA,
            pltpu.SemaphoreType.DMA,
        ],
        compiler_params=pltpu.CompilerParams(
            use_tc_tiling_on_sc=True, needs_layout_passes=False
        ),
    )(_sc_body)
    return run(table)


def kernel(q_len, k_len, table):
    del q_len, k_len
    return _expand(table)
